# baseline (speedup 1.0000x reference)
import functools

import jax
import jax.numpy as jnp
from jax.experimental import pallas as pl
from jax.experimental.pallas import tpu as pltpu

_TN = 512
_MC = 1024


def _chamfer_tile_kernel(a_ref, b_ref, xx_ref, yy_ref, xmin_ref, ymin8_ref):
    i = pl.program_id(1)
    TN = a_ref.shape[1]
    M = b_ref.shape[2]
    a = a_ref[0]
    xxc = xx_ref[0]
    xacc = None
    for j in range(M // _MC):
        sl = pl.ds(j * _MC, _MC)
        bj = b_ref[0, :, sl]
        g = jnp.dot(a, bj, preferred_element_type=jnp.float32)
        e = g + yy_ref[0, :, sl]
        xj = jnp.min(e.reshape(TN, _MC // 128, 128), axis=1)
        xacc = xj if xacc is None else jnp.minimum(xacc, xj)
        f = g + xxc
        yj = jnp.min(f.reshape(TN // 8, 8, _MC), axis=0)[None]

        @pl.when(i == 0)
        def _init():
            ymin8_ref[:, :, sl] = yj

        @pl.when(i > 0)
        def _acc():
            ymin8_ref[:, :, sl] = jnp.minimum(ymin8_ref[:, :, sl], yj)

    xmin_ref[0, 0, 0, :] = jnp.min(xacc, axis=1)


@functools.partial(jax.jit, static_argnames=())
def _chamfer(predict, target):
    B, N, _ = predict.shape
    _, M, _ = target.shape
    f32 = jnp.float32
    bf16 = jnp.bfloat16

    xx = jnp.sum(predict * predict, axis=-1, keepdims=True)
    amat = predict.astype(bf16)
    ty = target.transpose(0, 2, 1)
    yy = jnp.sum(ty * ty, axis=1, keepdims=True)
    bmat = (-2.0 * ty).astype(bf16)

    nb = N // _TN
    x_part, y_part8 = pl.pallas_call(
        _chamfer_tile_kernel,
        grid=(B, nb),
        in_specs=[
            pl.BlockSpec((1, _TN, 3), lambda b, i: (b, i, 0)),
            pl.BlockSpec((1, 3, M), lambda b, i: (b, 0, 0)),
            pl.BlockSpec((1, _TN, 1), lambda b, i: (b, i, 0)),
            pl.BlockSpec((1, 1, M), lambda b, i: (b, 0, 0)),
        ],
        out_specs=[
            pl.BlockSpec((1, 1, 1, _TN), lambda b, i: (b, i, 0, 0)),
            pl.BlockSpec((1, 8, M), lambda b, i: (b, 0, 0)),
        ],
        out_shape=[
            jax.ShapeDtypeStruct((B, nb, 1, _TN), f32),
            jax.ShapeDtypeStruct((B, 8, M), f32),
        ],
        compiler_params=pltpu.CompilerParams(
            dimension_semantics=("parallel", "arbitrary"),
        ),
    )(amat, bmat, xx, yy)
    x_near = x_part.reshape(B, N) + xx[:, :, 0]
    y_near = jnp.min(y_part8, axis=1) + yy[:, 0, :]
    return x_near.mean() + y_near.mean()


def kernel(predict, target):
    return _chamfer(predict, target)


# speedup vs baseline: 2.1675x; 2.1675x over previous
import functools

import jax
import jax.numpy as jnp
from jax.experimental import pallas as pl
from jax.experimental.pallas import tpu as pltpu

_TN = 512
_MC = 1024


def _chamfer_tile_kernel(a_ref, b_ref, xx_ref, yy_ref, xmin_ref, ymin8_ref):
    i = pl.program_id(1)
    TN = a_ref.shape[1]
    M = b_ref.shape[2]
    a = a_ref[0]
    xxc = xx_ref[0]
    xacc = None
    for j in range(M // _MC):
        sl = pl.ds(j * _MC, _MC)
        bj = b_ref[0, :, sl]
        g = jnp.dot(a, bj, preferred_element_type=jnp.float32)
        e = g + yy_ref[0, :, sl]
        xj = e[:, 0:128]
        for k in range(1, _MC // 128):
            xj = jnp.minimum(xj, e[:, k * 128:(k + 1) * 128])
        xacc = xj if xacc is None else jnp.minimum(xacc, xj)
        f = g + xxc
        yj = f[0:8, :]
        for r in range(1, TN // 8):
            yj = jnp.minimum(yj, f[r * 8:(r + 1) * 8, :])
        yj = yj[None]

        @pl.when(i == 0)
        def _init():
            ymin8_ref[:, :, sl] = yj

        @pl.when(i > 0)
        def _acc():
            ymin8_ref[:, :, sl] = jnp.minimum(ymin8_ref[:, :, sl], yj)

    xmin_ref[0, 0, 0, :] = jnp.min(xacc, axis=1)


@functools.partial(jax.jit, static_argnames=())
def _chamfer(predict, target):
    B, N, _ = predict.shape
    _, M, _ = target.shape
    f32 = jnp.float32
    bf16 = jnp.bfloat16

    xx = jnp.sum(predict * predict, axis=-1, keepdims=True)
    amat = predict.astype(bf16)
    ty = target.transpose(0, 2, 1)
    yy = jnp.sum(ty * ty, axis=1, keepdims=True)
    bmat = (-2.0 * ty).astype(bf16)

    nb = N // _TN
    x_part, y_part8 = pl.pallas_call(
        _chamfer_tile_kernel,
        grid=(B, nb),
        in_specs=[
            pl.BlockSpec((1, _TN, 3), lambda b, i: (b, i, 0)),
            pl.BlockSpec((1, 3, M), lambda b, i: (b, 0, 0)),
            pl.BlockSpec((1, _TN, 1), lambda b, i: (b, i, 0)),
            pl.BlockSpec((1, 1, M), lambda b, i: (b, 0, 0)),
        ],
        out_specs=[
            pl.BlockSpec((1, 1, 1, _TN), lambda b, i: (b, i, 0, 0)),
            pl.BlockSpec((1, 8, M), lambda b, i: (b, 0, 0)),
        ],
        out_shape=[
            jax.ShapeDtypeStruct((B, nb, 1, _TN), f32),
            jax.ShapeDtypeStruct((B, 8, M), f32),
        ],
        compiler_params=pltpu.CompilerParams(
            dimension_semantics=("parallel", "arbitrary"),
        ),
    )(amat, bmat, xx, yy)
    x_near = x_part.reshape(B, N) + xx[:, :, 0]
    y_near = jnp.min(y_part8, axis=1) + yy[:, 0, :]
    return x_near.mean() + y_near.mean()


def kernel(predict, target):
    return _chamfer(predict, target)


# speedup vs baseline: 3.0141x; 1.3906x over previous
import functools

import jax
import jax.numpy as jnp
from jax.experimental import pallas as pl
from jax.experimental.pallas import tpu as pltpu

_TN = 512
_MC = 1024


def _chamfer_tile_kernel(a_ref, b_ref, xx_ref, yy_ref, xmin_ref, ymin8_ref):
    i = pl.program_id(1)
    TN = a_ref.shape[1]
    M = b_ref.shape[2]
    a = a_ref[0]
    xxc = xx_ref[0]
    g = jnp.dot(a, b_ref[0], preferred_element_type=jnp.float32)
    e = g + yy_ref[0]
    xacc = e[:, 0:128]
    for k in range(1, M // 128):
        xacc = jnp.minimum(xacc, e[:, k * 128:(k + 1) * 128])
    f = g + xxc
    yj = f[0:8, :]
    for r in range(1, TN // 8):
        yj = jnp.minimum(yj, f[r * 8:(r + 1) * 8, :])
    yj = yj[None]

    @pl.when(i == 0)
    def _init():
        ymin8_ref[...] = yj

    @pl.when(i > 0)
    def _acc():
        ymin8_ref[...] = jnp.minimum(ymin8_ref[...], yj)

    xmin_ref[0, 0] = xacc


@functools.partial(jax.jit, static_argnames=())
def _chamfer(predict, target):
    B, N, _ = predict.shape
    _, M, _ = target.shape
    f32 = jnp.float32
    bf16 = jnp.bfloat16

    xx = jnp.sum(predict * predict, axis=-1, keepdims=True)
    amat = predict.astype(bf16)
    ty = target.transpose(0, 2, 1)
    yy = jnp.sum(ty * ty, axis=1, keepdims=True)
    bmat = (-2.0 * ty).astype(bf16)

    nb = N // _TN
    x_part, y_part8 = pl.pallas_call(
        _chamfer_tile_kernel,
        grid=(B, nb),
        in_specs=[
            pl.BlockSpec((1, _TN, 3), lambda b, i: (b, i, 0)),
            pl.BlockSpec((1, 3, M), lambda b, i: (b, 0, 0)),
            pl.BlockSpec((1, _TN, 1), lambda b, i: (b, i, 0)),
            pl.BlockSpec((1, 1, M), lambda b, i: (b, 0, 0)),
        ],
        out_specs=[
            pl.BlockSpec((1, 1, _TN, 128), lambda b, i: (b, i, 0, 0)),
            pl.BlockSpec((1, 8, M), lambda b, i: (b, 0, 0)),
        ],
        out_shape=[
            jax.ShapeDtypeStruct((B, nb, _TN, 128), f32),
            jax.ShapeDtypeStruct((B, 8, M), f32),
        ],
        compiler_params=pltpu.CompilerParams(
            dimension_semantics=("parallel", "arbitrary"),
        ),
    )(amat, bmat, xx, yy)
    x_near = jnp.min(x_part, axis=-1).reshape(B, N) + xx[:, :, 0]
    y_near = jnp.min(y_part8, axis=1) + yy[:, 0, :]
    return x_near.mean() + y_near.mean()


def kernel(predict, target):
    return _chamfer(predict, target)
